# Initial kernel scaffold; baseline (speedup 1.0000x reference)
import jax
import jax.numpy as jnp
from jax.experimental import pallas as pl


def kernel(X, A, W0, W1, W2, b_gcn, Wih0, Whh0, bih0, bhh0, Wih1, Whh1, bih1, bhh1, Wfc, bfc):
    raise NotImplementedError("write your pallas kernel here")


# speedup vs baseline: 751.0169x; 751.0169x over previous
import functools

import jax
import jax.numpy as jnp
from jax.experimental import pallas as pl
from jax.experimental.pallas import tpu as pltpu

TH = 12
TP = 3
HID = 32
B = 8
N = 512
BN = B * N


def _cheb_kernel(a_ref, at_ref, x_ref, bw0_ref, bw1_ref, bw2_ref, bg_ref, hn_ref):
    a = a_ref[...]
    at = at_ref[...]
    x = x_ref[...]
    deg = jnp.sum(a, axis=1, keepdims=True)
    dinv = jnp.where(deg > 0, jax.lax.rsqrt(deg), 0.0)
    t0 = x
    t1 = -(dinv * jnp.dot(at, dinv * t0, preferred_element_type=jnp.float32))
    t2 = -2.0 * (dinv * jnp.dot(at, dinv * t1, preferred_element_type=jnp.float32)) - t0
    hn = (jnp.dot(t0, bw0_ref[...], preferred_element_type=jnp.float32)
          + jnp.dot(t1, bw1_ref[...], preferred_element_type=jnp.float32)
          + jnp.dot(t2, bw2_ref[...], preferred_element_type=jnp.float32)
          + bg_ref[...])
    hn_ref[...] = hn


def _lstm_kernel(xr_ref, hr_ref, wx0_ref, wh0_ref, b0_ref, wx1_ref, wh1_ref,
                 b1_ref, wfc_ref, out_ref, h0s_ref):
    xr = xr_ref[...]
    hr = hr_ref[...]
    wx0 = wx0_ref[...]
    wh0 = wh0_ref[...]
    b0 = b0_ref[...]
    wx1 = wx1_ref[...]
    wh1 = wh1_ref[...]
    b1 = b1_ref[...]

    def gates_to_hc(gates, c):
        i = jax.nn.sigmoid(gates[:, 0 * HID:1 * HID])
        f = jax.nn.sigmoid(gates[:, 1 * HID:2 * HID])
        g = jnp.tanh(gates[:, 2 * HID:3 * HID])
        o = jax.nn.sigmoid(gates[:, 3 * HID:4 * HID])
        c = f * c + i * g
        h = o * jnp.tanh(c)
        return h, c

    h = jnp.zeros((BN, HID), jnp.float32)
    c = jnp.zeros((BN, HID), jnp.float32)
    for t in range(TH):
        src = xr if t < TH // 2 else hr
        j = (2 * t) % TH
        xpair = src[:, j:j + 2]
        gates = (jnp.dot(xpair, wx0, preferred_element_type=jnp.float32)
                 + jnp.dot(h, wh0, preferred_element_type=jnp.float32) + b0)
        h, c = gates_to_hc(gates, c)
        h0s_ref[:, t * HID:(t + 1) * HID] = h

    h = jnp.zeros((BN, HID), jnp.float32)
    c = jnp.zeros((BN, HID), jnp.float32)
    lasts = []
    for t in range(TH):
        xt = h0s_ref[:, t * HID:(t + 1) * HID]
        gates = (jnp.dot(xt, wx1, preferred_element_type=jnp.float32)
                 + jnp.dot(h, wh1, preferred_element_type=jnp.float32) + b1)
        h, c = gates_to_hc(gates, c)
        if t >= TH - TP:
            lasts.append(h)

    hcat = jnp.concatenate(lasts, axis=1)
    out_ref[...] = jnp.dot(hcat, wfc_ref[...], preferred_element_type=jnp.float32)


@functools.partial(jax.jit, static_argnums=())
def kernel(X, A, W0, W1, W2, b_gcn, Wih0, Whh0, bih0, bhh0,
           Wih1, Whh1, bih1, bhh1, Wfc, bfc):
    f32 = jnp.float32
    Xn = X.transpose(1, 0, 2).reshape(N, B * TH)
    At = A.T
    eyeB = jnp.eye(B, dtype=f32)
    BW0 = jnp.kron(eyeB, W0)
    BW1 = jnp.kron(eyeB, W1)
    BW2 = jnp.kron(eyeB, W2)
    bg = jnp.tile(b_gcn, B)[None, :]

    hn = pl.pallas_call(
        _cheb_kernel,
        out_shape=jax.ShapeDtypeStruct((N, B * TH), f32),
    )(A, At, Xn, BW0, BW1, BW2, bg)

    Xr = Xn.reshape(BN, TH)
    Hr = hn.reshape(BN, TH)
    wx0 = Wih0.T
    wh0 = Whh0.T
    b0 = (bih0 + bhh0)[None, :]
    wx1 = Wih1.T
    wh1 = Whh1.T
    b1 = (bih1 + bhh1)[None, :]
    wfc_blk = jnp.kron(jnp.eye(TP, dtype=f32), Wfc.T)

    out = pl.pallas_call(
        _lstm_kernel,
        out_shape=jax.ShapeDtypeStruct((BN, TP), f32),
        scratch_shapes=[pltpu.VMEM((BN, TH * HID), f32)],
    )(Xr, Hr, wx0, wh0, b0, wx1, wh1, b1, wfc_blk)

    out = out + bfc[0]
    return out.reshape(N, B, TP).transpose(1, 0, 2)


# speedup vs baseline: 1357.7166x; 1.8078x over previous
import numpy as np

import jax
import jax.numpy as jnp
from jax.experimental import pallas as pl
from jax.experimental.pallas import tpu as pltpu

TH = 12
TP = 3
HID = 32
B = 8
N = 512
BN = B * N
PK = 4
PR = BN // PK
G4 = 4 * HID * PK

_SEL = np.zeros((TH, 2, 2 * TH * PK, PK), np.float32)
for _t in range(TH):
    for _j in range(2):
        for _s in range(PK):
            _SEL[_t, _j, 2 * TH * _s + 2 * _t + _j, _s] = 1.0
_EYE4 = np.eye(PK, dtype=np.float32)


def _cheb_kernel(a_ref, at_ref, x_ref, bw0_ref, bw1_ref, bw2_ref, bg_ref, hn_ref):
    a = a_ref[...]
    at = at_ref[...]
    x = x_ref[...]
    deg = jnp.sum(a, axis=1, keepdims=True)
    dinv = jnp.where(deg > 0, jax.lax.rsqrt(deg), 0.0)
    t0 = x
    t1 = -(dinv * jnp.dot(at, dinv * t0, preferred_element_type=jnp.float32))
    t2 = -2.0 * (dinv * jnp.dot(at, dinv * t1, preferred_element_type=jnp.float32)) - t0
    hn = (jnp.dot(t0, bw0_ref[...], preferred_element_type=jnp.float32)
          + jnp.dot(t1, bw1_ref[...], preferred_element_type=jnp.float32)
          + jnp.dot(t2, bw2_ref[...], preferred_element_type=jnp.float32)
          + bg_ref[...])
    hn_ref[...] = hn


def _lstm_kernel(vp_ref, selw_ref, bwh0_ref, b0_ref, bwx1_ref, bwh1_ref,
                 b1_ref, bwfc_ref, out_ref, h0s_ref):
    vp = vp_ref[...]
    bwh0 = bwh0_ref[...]
    b0 = b0_ref[...]
    bwx1 = bwx1_ref[...]
    bwh1 = bwh1_ref[...]
    b1 = b1_ref[...]
    HP = HID * PK

    def gates_to_hc(gates, c):
        i = jax.nn.sigmoid(gates[:, 0 * HP:1 * HP])
        f = jax.nn.sigmoid(gates[:, 1 * HP:2 * HP])
        g = jnp.tanh(gates[:, 2 * HP:3 * HP])
        o = jax.nn.sigmoid(gates[:, 3 * HP:4 * HP])
        c = f * c + i * g
        h = o * jnp.tanh(c)
        return h, c

    h = jnp.zeros((PR, HP), jnp.float32)
    c = jnp.zeros((PR, HP), jnp.float32)
    for t in range(TH):
        gates = (jnp.dot(vp, selw_ref[t], preferred_element_type=jnp.float32)
                 + jnp.dot(h, bwh0, preferred_element_type=jnp.float32) + b0)
        h, c = gates_to_hc(gates, c)
        h0s_ref[:, t * HP:(t + 1) * HP] = h

    h = jnp.zeros((PR, HP), jnp.float32)
    c = jnp.zeros((PR, HP), jnp.float32)
    for t in range(TH):
        xt = h0s_ref[:, t * HP:(t + 1) * HP]
        gates = (jnp.dot(xt, bwx1, preferred_element_type=jnp.float32)
                 + jnp.dot(h, bwh1, preferred_element_type=jnp.float32) + b1)
        h, c = gates_to_hc(gates, c)
        if t >= TH - TP:
            k = t - (TH - TP)
            out_ref[:, k * PK:(k + 1) * PK] = jnp.dot(
                h, bwfc_ref[...], preferred_element_type=jnp.float32)


def _pack_rec(W):
    wt = W.T.reshape(HID, 4, HID)
    return jnp.einsum('st,kgh->skgth', _EYE4, wt).reshape(PK * HID, G4)


def _pack_bias(b):
    return jnp.broadcast_to(b.reshape(4, 1, HID), (4, PK, HID)).reshape(1, G4)


def kernel(X, A, W0, W1, W2, b_gcn, Wih0, Whh0, bih0, bhh0,
           Wih1, Whh1, bih1, bhh1, Wfc, bfc):
    f32 = jnp.float32
    Xn = X.transpose(1, 0, 2).reshape(N, B * TH)
    At = A.T
    eyeB = jnp.eye(B, dtype=f32)
    BW0 = jnp.kron(eyeB, W0)
    BW1 = jnp.kron(eyeB, W1)
    BW2 = jnp.kron(eyeB, W2)
    bg = jnp.tile(b_gcn, B)[None, :]

    hn = pl.pallas_call(
        _cheb_kernel,
        out_shape=jax.ShapeDtypeStruct((N, B * TH), f32),
    )(A, At, Xn, BW0, BW1, BW2, bg)

    Xr = Xn.reshape(BN, TH)
    Vp = jnp.concatenate([Xr, hn.reshape(BN, TH)], axis=-1).reshape(PR, PK * 2 * TH)
    wj = Wih0.T.reshape(2, 4, HID)
    Q = jnp.einsum('st,jgh->jsgth', _EYE4, wj).reshape(2, PK, G4)
    selw = jnp.einsum('tjab,jbc->tac', jnp.asarray(_SEL), Q)
    bwh0 = _pack_rec(Whh0)
    b0 = _pack_bias(bih0 + bhh0)
    bwx1 = _pack_rec(Wih1)
    bwh1 = _pack_rec(Whh1)
    b1 = _pack_bias(bih1 + bhh1)
    bwfc = jnp.einsum('st,k->skt', _EYE4, Wfc[0]).reshape(PK * HID, PK)

    out = pl.pallas_call(
        _lstm_kernel,
        out_shape=jax.ShapeDtypeStruct((PR, TP * PK), f32),
        scratch_shapes=[pltpu.VMEM((PR, TH * HID * PK), f32)],
    )(Vp, selw, bwh0, b0, bwx1, bwh1, b1, bwfc)

    out = (out + bfc[0]).reshape(PR, TP, PK).transpose(0, 2, 1).reshape(BN, TP)
    return out.reshape(N, B, TP).transpose(1, 0, 2)
